# baseline (speedup 1.0000x reference)
import jax
import jax.numpy as jnp
from jax.experimental import pallas as pl
from jax.experimental.pallas import tpu as pltpu

CAP = 8192
G = 8
HD = 128
NEW = 32
BLK = 2048
NBLK = CAP // BLK


def _extend_one_body(cl_ref, buf, new_rows, out):
    i = pl.program_id(0)
    blk_start = i * BLK
    out[...] = buf[...]

    cl = cl_ref[0]

    @pl.when(jnp.logical_and(cl + NEW > blk_start, cl < blk_start + BLK))
    def _():
        def body(r, carry):
            dest = cl + r - blk_start

            @pl.when(jnp.logical_and(dest >= 0, dest < BLK))
            def _():
                out[pl.ds(dest, 1)] = new_rows[pl.ds(r, 1)]

            return carry

        jax.lax.fori_loop(0, NEW, body, 0)


def _extend_one(cl1, buf, new_rows):
    return pl.pallas_call(
        _extend_one_body,
        grid=(NBLK,),
        in_specs=[
            pl.BlockSpec(memory_space=pltpu.SMEM),
            pl.BlockSpec((BLK, G, HD), lambda i: (i, 0, 0)),
            pl.BlockSpec((NEW, G, HD), lambda i: (0, 0, 0)),
        ],
        out_specs=pl.BlockSpec((BLK, G, HD), lambda i: (i, 0, 0)),
        out_shape=jax.ShapeDtypeStruct((CAP, G, HD), jnp.float32),
        compiler_params=pltpu.CompilerParams(
            dimension_semantics=("arbitrary",),
        ),
    )(cl1, buf, new_rows)


def kernel(keys, values, current_length, new_keys, new_values):
    cl1 = current_length.reshape(1)
    out_k = _extend_one(cl1, keys, new_keys)
    out_v = _extend_one(cl1, values, new_values)
    return (out_k, out_v, current_length + NEW)
